# Initial kernel scaffold; baseline (speedup 1.0000x reference)
import jax
import jax.numpy as jnp
from jax.experimental import pallas as pl


def kernel(node_feats, edge_feats, edge_index, params):
    raise NotImplementedError("write your pallas kernel here")


# speedup vs baseline: 6.4037x; 6.4037x over previous
import functools

import jax
import jax.numpy as jnp
from jax import lax
from jax.experimental import pallas as pl
from jax.experimental.pallas import tpu as pltpu
from jax.experimental.pallas import tpu_sc as plsc

EN = 320000
NN = 10000
DD = 128
NC = 2
NS = 16
NW = NC * NS
CHUNK = EN // NW
CHUNK_A = EN // NS
CAPV = 32
NV = NS * CAPV
CAPVB = NV // NW
P0SLOT = NV
VP = NV + 128
CAPR = 128
RTOT = NW * CAPR
NPAD = 10240
GC = 64
SLACK = 64


def _wid():
    return lax.axis_index("s") * NC + lax.axis_index("c")


def _sc_a_body(dst_hbm, src_hbm, v_hbm, dst_v, src_v, vbuf):
    w = _wid()
    base = w * CHUNK
    pltpu.sync_copy(dst_hbm.at[pl.ds(base, CHUNK)], dst_v)
    pltpu.sync_copy(src_hbm.at[pl.ds(base, CHUNK)], src_v)
    for k in range((CAPVB + SLACK) // 16):
        vbuf[pl.ds(k * 16, 16)] = jnp.full((16,), -1, jnp.int32)

    def body(i, cnt):
        d = dst_v[pl.ds(i * 16, 16)]
        m = d == 0
        c = jnp.sum(jnp.where(m, 1, 0).astype(jnp.int32))

        @pl.when(c > 0)
        def _():
            sv = src_v[pl.ds(i * 16, 16)]
            plsc.store_compressed(vbuf.at[pl.ds(cnt, 16)], sv, mask=m)

        return cnt + c

    lax.fori_loop(0, CHUNK // 16, body, 0)
    pltpu.sync_copy(vbuf.at[pl.ds(0, CAPVB)], v_hbm.at[pl.ds(w * CAPVB, CAPVB)])


def _sc_a(dst, src):
    mesh = plsc.VectorSubcoreMesh(core_axis_name="c", subcore_axis_name="s")
    f = functools.partial(
        pl.kernel,
        mesh=mesh,
        out_type=jax.ShapeDtypeStruct((NV,), jnp.int32),
        scratch_types=[
            pltpu.VMEM((CHUNK,), jnp.int32),
            pltpu.VMEM((CHUNK,), jnp.int32),
            pltpu.VMEM((CAPVB + SLACK,), jnp.int32),
        ],
    )(_sc_a_body)
    return f(dst, src)


def _t1_body(v_ref, tbl_ref):
    v = v_ref[...]
    jio = lax.broadcasted_iota(jnp.int32, (NV, 1), 0)
    big = jnp.int32(1 << 20)
    cols = []
    for t in range(NPAD // 128):
        nt = lax.broadcasted_iota(jnp.int32, (1, 128), 1) + t * 128
        val = jnp.where(v == nt, jio, big)
        cols.append(jnp.min(val, axis=0, keepdims=True))
    tbl = jnp.concatenate(cols, axis=0)
    node = (
        lax.broadcasted_iota(jnp.int32, (NPAD // 128, 128), 0) * 128
        + lax.broadcasted_iota(jnp.int32, (NPAD // 128, 128), 1)
    )
    tbl = jnp.where(node == 0, jnp.minimum(tbl, P0SLOT), tbl)
    tbl_ref[...] = jnp.where(tbl >= big, -1, tbl)


def _t1(vvals2d):
    return pl.pallas_call(
        _t1_body,
        out_shape=jax.ShapeDtypeStruct((NPAD // 128, 128), jnp.int32),
    )(vvals2d)


def _sc_b_body(
    dst_hbm, src_hbm, tbl_hbm, vvals_hbm, x_hbm, ef_hbm,
    pvd_hbm, pvs_hbm, dst2_hbm, xsrc_hbm, xdst_hbm, efr_hbm, xv_hbm,
    tbl_v, dst_v, src_v, eidb, srcb, dstb, pvdb, pvsb,
    idx16, rows16, rowbuf, sem,
):
    w = _wid()
    base = w * CHUNK
    pltpu.sync_copy(tbl_hbm.at[pl.ds(0, NPAD)], tbl_v)
    pltpu.sync_copy(dst_hbm.at[pl.ds(base, CHUNK)], dst_v)
    pltpu.sync_copy(src_hbm.at[pl.ds(base, CHUNK)], src_v)
    for k in range((CAPR + SLACK) // 16):
        z = jnp.full((16,), 0, jnp.int32)
        eidb[pl.ds(k * 16, 16)] = z
        srcb[pl.ds(k * 16, 16)] = z
        dstb[pl.ds(k * 16, 16)] = z
        pvdb[pl.ds(k * 16, 16)] = jnp.full((16,), -1, jnp.int32)
        pvsb[pl.ds(k * 16, 16)] = jnp.full((16,), -1, jnp.int32)

    def body(i, cnt):
        d = dst_v[pl.ds(i * 16, 16)]
        pvd = d * 0
        m = pvd >= 0
        c = jnp.sum(jnp.where(m, 1, 0).astype(jnp.int32))

        @pl.when(c > 0)
        def _():
            s = src_v[pl.ds(i * 16, 16)]
            pvs = s * 0
            eid = s
            plsc.store_compressed(eidb.at[pl.ds(cnt, 16)], eid, mask=m)
            plsc.store_compressed(srcb.at[pl.ds(cnt, 16)], s, mask=m)
            plsc.store_compressed(dstb.at[pl.ds(cnt, 16)], d, mask=m)
            plsc.store_compressed(pvdb.at[pl.ds(cnt, 16)], pvd, mask=m)
            plsc.store_compressed(pvsb.at[pl.ds(cnt, 16)], pvs, mask=m)

        return cnt + c

    lax.fori_loop(0, CHUNK // 16, body, 0)

    for r in range(CAPR // 128):
        pltpu.sync_copy(pvdb.at[pl.ds(r * 128, 128)],
                        pvd_hbm.at[w * (CAPR // 128) + r])
        pltpu.sync_copy(pvsb.at[pl.ds(r * 128, 128)],
                        pvs_hbm.at[w * (CAPR // 128) + r])
        pltpu.sync_copy(dstb.at[pl.ds(r * 128, 128)],
                        dst2_hbm.at[w * (CAPR // 128) + r])

    return
    for cix in range(CAPR // GC):
        off = cix * GC
        pltpu.async_copy(x_hbm.at[srcb.at[pl.ds(off, GC)]], rowbuf, sem).wait()
        pltpu.sync_copy(rowbuf, xsrc_hbm.at[pl.ds(w * CAPR + off, GC)])
        pltpu.async_copy(x_hbm.at[dstb.at[pl.ds(off, GC)]], rowbuf, sem).wait()
        pltpu.sync_copy(rowbuf, xdst_hbm.at[pl.ds(w * CAPR + off, GC)])
        pltpu.async_copy(ef_hbm.at[eidb.at[pl.ds(off, GC)]], rowbuf, sem).wait()
        pltpu.sync_copy(rowbuf, efr_hbm.at[pl.ds(w * CAPR + off, GC)])

    pltpu.sync_copy(vvals_hbm.at[pl.ds(w * CAPVB, CAPVB)], idx16)
    idx16[...] = jnp.maximum(idx16[...], 0)
    pltpu.async_copy(x_hbm.at[idx16], rows16, sem).wait()
    pltpu.sync_copy(rows16, xv_hbm.at[pl.ds(w * CAPVB, CAPVB)])

    @pl.when(w < (VP - NV) // CAPVB)
    def _():
        idx16[...] = jnp.full((16,), 0, jnp.int32)
        pltpu.async_copy(x_hbm.at[idx16], rows16, sem).wait()
        pltpu.sync_copy(rows16, xv_hbm.at[pl.ds(NV + w * CAPVB, CAPVB)])


def _sc_b(dst, src, tbl, vvals, x, ef):
    mesh = plsc.VectorSubcoreMesh(core_axis_name="c", subcore_axis_name="s")
    f = functools.partial(
        pl.kernel,
        mesh=mesh,
        out_type=[
            jax.ShapeDtypeStruct((RTOT // 128, 128), jnp.int32),
            jax.ShapeDtypeStruct((RTOT // 128, 128), jnp.int32),
            jax.ShapeDtypeStruct((RTOT // 128, 128), jnp.int32),
            jax.ShapeDtypeStruct((RTOT, DD), jnp.float32),
            jax.ShapeDtypeStruct((RTOT, DD), jnp.float32),
            jax.ShapeDtypeStruct((RTOT, DD), jnp.float32),
            jax.ShapeDtypeStruct((VP, DD), jnp.float32),
        ],
        scratch_types=[
            pltpu.VMEM((NPAD,), jnp.int32),
            pltpu.VMEM((CHUNK,), jnp.int32),
            pltpu.VMEM((CHUNK,), jnp.int32),
            pltpu.VMEM((CAPR + SLACK,), jnp.int32),
            pltpu.VMEM((CAPR + SLACK,), jnp.int32),
            pltpu.VMEM((CAPR + SLACK,), jnp.int32),
            pltpu.VMEM((CAPR + SLACK,), jnp.int32),
            pltpu.VMEM((CAPR + SLACK,), jnp.int32),
            pltpu.VMEM((CAPVB,), jnp.int32),
            pltpu.VMEM((CAPVB, DD), jnp.float32),
            pltpu.VMEM((GC, DD), jnp.float32),
            pltpu.SemaphoreType.DMA,
        ],
    )(_sc_b_body)
    return f(dst, src, tbl, vvals, x, ef)


NCH = 8
CH = RTOT // NCH


def _dotT(a, b):
    return lax.dot_general(
        a, b, (((0,), (0,)), ((), ())), preferred_element_type=jnp.float32
    )


def _leaky(v):
    return jnp.where(v >= 0, v, 0.2 * v)


def _c_body(
    pvd_ref, pvs_ref, dst_ref, xsrc_ref, xdst_ref, efr_ref, xv_ref, x0_ref,
    wni1, wnj1, wfij1, wnode1, bni1, bnj1, bfij1, bnode1, attn1,
    wni2, wnj2, wfij2, wnode2, bni2, bnj2, bfij2, bnode2, attn2,
    out_ref,
):
    dot = functools.partial(jnp.dot, preferred_element_type=jnp.float32)
    pvd = pvd_ref[...]
    pvs = pvs_ref[...]
    xsrc = xsrc_ref[...]
    xdst = xdst_ref[...]
    efr = efr_ref[...]
    xv = xv_ref[...]
    dstv = dst_ref[...]

    f1 = _leaky(
        dot(xsrc, wni1[...]) + dot(xdst, wnj1[...]) + dot(efr, wfij1[...])
        + bni1[...] + bnj1[...] + bfij1[...]
    )
    e1 = dot(f1, attn1[...])
    gmax = jnp.max(e1)
    ee1 = jnp.exp(e1 - gmax)
    ft1 = dot(xsrc, wnode1[...]) + bnode1[...]

    iov = lax.broadcasted_iota(jnp.int32, (CH, VP), 1)

    def onehot(p, c):
        return (p[c * CH:(c + 1) * CH] == iov).astype(jnp.float32)

    denom = jnp.zeros((VP, 1), jnp.float32)
    for c in range(NCH):
        denom = denom + _dotT(onehot(pvd, c), ee1[c * CH:(c + 1) * CH])
    rst1 = jnp.zeros((VP, DD), jnp.float32)
    for c in range(NCH):
        oc = onehot(pvd, c)
        dg = dot(oc, denom)
        a1 = ee1[c * CH:(c + 1) * CH] / jnp.maximum(dg, 1e-9)
        rst1 = rst1 + _dotT(oc, a1 * ft1[c * CH:(c + 1) * CH])

    h0v = jnp.maximum(rst1, 0.0) + xv
    fniv = dot(h0v, wni2[...]) + bni2[...]
    fnjv = dot(h0v, wnj2[...]) + bnj2[...]
    fndv = dot(h0v, wnode2[...]) + bnode2[...]
    mask2 = (dstv == 0) & (pvd >= 0)
    onep0 = jnp.zeros((1, VP), jnp.float32)
    for c in range(NCH):
        sel = onehot(pvd, c) * jnp.where(mask2[c * CH:(c + 1) * CH], 1.0, 0.0)
        onep0 = jnp.maximum(onep0, jnp.max(sel, axis=0, keepdims=True))
    fnj2 = dot(onep0, fnjv)

    h1 = jnp.maximum(f1, 0.0) + efr
    ffij2 = dot(h1, wfij2[...]) + bfij2[...]

    e2s = []
    ft2s = []
    for c in range(NCH):
        os = onehot(pvs, c)
        f2 = _leaky(dot(os, fniv) + fnj2 + ffij2[c * CH:(c + 1) * CH])
        e2s.append(dot(f2, attn2[...]))
        ft2s.append(dot(os, fndv))
    e2 = jnp.concatenate(e2s, axis=0)
    ft2 = jnp.concatenate(ft2s, axis=0)

    m2 = jnp.max(jnp.where(mask2, e2, -1e30))
    ee2 = jnp.where(mask2, jnp.exp(e2 - m2), 0.0)
    d2 = jnp.sum(ee2)
    a2 = ee2 / jnp.maximum(d2, 1e-9)
    out_ref[...] = _dotT(a2, ft2) + x0_ref[...]


def _conv_c(pvd, pvs, dstv, xsrc, xdst, efr, xv, x0, p1, p2):
    args = [pvd, pvs, dstv, xsrc, xdst, efr, xv, x0]
    for p in (p1, p2):
        args += [
            p["W_ni"], p["W_nj"], p["W_fij"], p["W_node"],
            p["b_ni"].reshape(1, DD), p["b_nj"].reshape(1, DD),
            p["b_fij"].reshape(1, DD), p["b_node"].reshape(1, DD),
            p["attn"].reshape(DD, 1),
        ]
    return pl.pallas_call(
        _c_body,
        out_shape=jax.ShapeDtypeStruct((1, DD), jnp.float32),
    )(*args)


@jax.jit
def kernel(node_feats, edge_feats, edge_index, params):
    x = node_feats
    src = edge_index[0].astype(jnp.int32)
    dst = edge_index[1].astype(jnp.int32)
    big = jnp.int32(1 << 20)

    m0 = dst == 0
    i0 = jnp.nonzero(m0, size=NV, fill_value=0)[0]
    vvals = jnp.where(
        jnp.arange(NV) < jnp.sum(jnp.where(m0, 1, 0)), src[i0], -1
    )

    pos = jnp.where(vvals >= 0, jnp.arange(NV, dtype=jnp.int32), big)
    nid = jnp.arange(NPAD, dtype=jnp.int32)
    tbl = jnp.min(
        jnp.where(vvals[:, None] == nid[None, :], pos[:, None], big), axis=0
    )
    tbl = jnp.where(nid == 0, jnp.minimum(tbl, P0SLOT), tbl)
    p0 = tbl[0]
    tbl = jnp.where(tbl >= big, -1, tbl)

    member = tbl[dst] >= 0
    idx = jnp.nonzero(member, size=RTOT, fill_value=0)[0]
    valid = jnp.arange(RTOT) < jnp.sum(jnp.where(member, 1, 0))
    srcv = src[idx]
    pvd = jnp.where(valid, tbl[dst[idx]], -1)[:, None]
    pvs = jnp.where(valid, tbl[srcv], -1)[:, None]
    dstv = jnp.where(valid, dst[idx], -1)[:, None]
    xsrc = x[srcv]
    xdst = x[dst[idx]]
    efr = edge_feats[idx]
    xv = jnp.concatenate(
        [x[jnp.maximum(vvals, 0)],
         jnp.broadcast_to(x[0:1], (VP - NV, DD))], axis=0
    )

    return _conv_c(
        pvd, pvs, dstv, xsrc, xdst, efr, xv, x[0:1],
        params["conv1"], params["conv2"]
    )
